# Initial kernel scaffold; baseline (speedup 1.0000x reference)
import jax
import jax.numpy as jnp
from jax.experimental import pallas as pl


def kernel(x, conv1_w, conv1_b, conv2_w, conv2_b, fc1_w, fc1_b, fc2_w, fc2_b, fc3_w, fc3_b):
    raise NotImplementedError("write your pallas kernel here")


# speedup vs baseline: 2.5024x; 2.5024x over previous
import jax
import jax.numpy as jnp
from jax.experimental import pallas as pl
from jax.experimental.pallas import tpu as pltpu

_H1, _W1, _C0, _C1 = 128, 128, 3, 16
_H2, _W2, _C2 = 64, 64, 32
_P1 = _H1 * _W1
_P2 = _H2 * _W2
_P3 = (_H2 // 2) * (_W2 // 2)


def _im2col_dot(xpad, w, *, Cin, W, P):
    col = jax.lax.broadcasted_iota(jnp.int32, (1, P), 1) & (W - 1)
    mask_l = col == 0
    mask_r = col == (W - 1)
    taps = []
    for kh in range(3):
        for kw in range(3):
            start = 2 * W + (kh - 1) * W + (kw - 1)
            s = xpad[:, start:start + P]
            if kw == 0:
                s = jnp.where(mask_l, 0.0, s)
            elif kw == 2:
                s = jnp.where(mask_r, 0.0, s)
            taps.append(s)
    patch = jnp.concatenate(taps, axis=0)
    return jnp.dot(w, patch, preferred_element_type=jnp.float32)


def _pool_bias_relu(conv, b, *, C, H, W):
    c = conv.reshape(C, H // 2, 2, W)
    hp = jnp.max(c, axis=2)
    sh = jnp.concatenate([hp[:, :, 1:], hp[:, :, :1]], axis=2)
    m = jnp.maximum(hp, sh).reshape(C * (H // 2), W)
    sel = (jax.lax.broadcasted_iota(jnp.int32, (W, W // 2), 0)
           == 2 * jax.lax.broadcasted_iota(jnp.int32, (W, W // 2), 1)
           ).astype(jnp.float32)
    wp = jnp.dot(m, sel, preferred_element_type=jnp.float32)
    y = jnp.maximum(wp.reshape(C, H // 2, W // 2) + b.reshape(C, 1, 1), 0.0)
    return y.reshape(C, (H // 2) * (W // 2))


def _convs_kernel(x_ref, w1_ref, b1_ref, w2_ref, b2_ref, o_ref,
                  xpad_ref, ypad_ref):
    xpad_ref[:, :2 * _W1] = jnp.zeros((_C0, 2 * _W1), jnp.float32)
    xpad_ref[:, 2 * _W1 + _P1:] = jnp.zeros((_C0, 2 * _W1), jnp.float32)
    xpad_ref[:, 2 * _W1:2 * _W1 + _P1] = x_ref[0]
    conv1 = _im2col_dot(xpad_ref, w1_ref[...], Cin=_C0, W=_W1, P=_P1)
    y1 = _pool_bias_relu(conv1, b1_ref[...], C=_C1, H=_H1, W=_W1)

    ypad_ref[:, :2 * _W2] = jnp.zeros((_C1, 2 * _W2), jnp.float32)
    ypad_ref[:, 2 * _W2 + _P2:] = jnp.zeros((_C1, 2 * _W2), jnp.float32)
    ypad_ref[:, 2 * _W2:2 * _W2 + _P2] = y1
    conv2 = _im2col_dot(ypad_ref, w2_ref[...], Cin=_C1, W=_W2, P=_P2)
    o_ref[0] = _pool_bias_relu(conv2, b2_ref[...], C=_C2, H=_H2, W=_W2)


def _fused_convs(x_flat, w1, b1, w2, b2):
    B = x_flat.shape[0]
    return pl.pallas_call(
        _convs_kernel,
        out_shape=jax.ShapeDtypeStruct((B, _C2, _P3), jnp.float32),
        grid=(B,),
        in_specs=[
            pl.BlockSpec((1, _C0, _P1), lambda b: (b, 0, 0)),
            pl.BlockSpec((_C1, 9 * _C0), lambda b: (0, 0)),
            pl.BlockSpec((_C1, 1), lambda b: (0, 0)),
            pl.BlockSpec((_C2, 9 * _C1), lambda b: (0, 0)),
            pl.BlockSpec((_C2, 1), lambda b: (0, 0)),
        ],
        out_specs=pl.BlockSpec((1, _C2, _P3), lambda b: (b, 0, 0)),
        scratch_shapes=[
            pltpu.VMEM((_C0, _P1 + 4 * _W1), jnp.float32),
            pltpu.VMEM((_C1, _P2 + 4 * _W2), jnp.float32),
        ],
        compiler_params=pltpu.CompilerParams(
            dimension_semantics=("arbitrary",)),
    )(x_flat, w1, b1, w2, b2)


def _fc_kernel(x_ref, w1_ref, b1_ref, w2_ref, b2_ref, w3_ref, b3_ref,
               o_ref, acc_ref):
    k = pl.program_id(0)

    @pl.when(k == 0)
    def _():
        acc_ref[...] = jnp.zeros_like(acc_ref)

    acc_ref[...] += jnp.dot(x_ref[...], w1_ref[...],
                            preferred_element_type=jnp.float32)

    @pl.when(k == pl.num_programs(0) - 1)
    def _():
        h1 = acc_ref[...] + b1_ref[...]
        h2 = jnp.dot(h1, w2_ref[...],
                     preferred_element_type=jnp.float32) + b2_ref[...]
        o_ref[...] = (jnp.dot(h2, w3_ref[...],
                              preferred_element_type=jnp.float32)
                      + b3_ref[...]).astype(o_ref.dtype)


def _fc_head(x_flat, w1, b1, w2, b2, w3, b3, *, tk=8192):
    B, K = x_flat.shape
    H1, H2, NC = w1.shape[1], w2.shape[1], w3.shape[1]
    return pl.pallas_call(
        _fc_kernel,
        out_shape=jax.ShapeDtypeStruct((B, NC), jnp.float32),
        grid=(K // tk,),
        in_specs=[
            pl.BlockSpec((B, tk), lambda k: (0, k)),
            pl.BlockSpec((tk, H1), lambda k: (k, 0)),
            pl.BlockSpec((1, H1), lambda k: (0, 0)),
            pl.BlockSpec((H1, H2), lambda k: (0, 0)),
            pl.BlockSpec((1, H2), lambda k: (0, 0)),
            pl.BlockSpec((H2, NC), lambda k: (0, 0)),
            pl.BlockSpec((1, NC), lambda k: (0, 0)),
        ],
        out_specs=pl.BlockSpec((B, NC), lambda k: (0, 0)),
        scratch_shapes=[pltpu.VMEM((B, H1), jnp.float32)],
        compiler_params=pltpu.CompilerParams(
            dimension_semantics=("arbitrary",)),
    )(x_flat, w1, b1, w2, b2, w3, b3)


@jax.jit
def kernel(x, conv1_w, conv1_b, conv2_w, conv2_b,
           fc1_w, fc1_b, fc2_w, fc2_b, fc3_w, fc3_b):
    B = x.shape[0]
    x_flat = x.astype(jnp.float32).reshape(B, _C0, _P1)
    y2 = _fused_convs(x_flat, conv1_w, conv1_b, conv2_w, conv2_b)
    flat = y2.reshape(B, _C2 * _P3)
    return _fc_head(flat, fc1_w, fc1_b, fc2_w, fc2_b, fc3_w, fc3_b)


# speedup vs baseline: 4.5895x; 1.8341x over previous
import jax
import jax.numpy as jnp
from jax.experimental import pallas as pl
from jax.experimental.pallas import tpu as pltpu

_H1, _W1, _C0, _C1 = 128, 128, 3, 16
_H2, _W2, _C2 = 64, 64, 32
_P1 = _H1 * _W1
_P2 = _H2 * _W2
_P3 = (_H2 // 2) * (_W2 // 2)


def _im2col_dot(xpad, w, *, Cin, W, P):
    col = jax.lax.broadcasted_iota(jnp.int32, (1, P), 1) & (W - 1)
    mask_l = col == 0
    mask_r = col == (W - 1)
    taps = []
    for kh in range(3):
        for kw in range(3):
            start = 2 * W + (kh - 1) * W + (kw - 1)
            s = xpad[:, start:start + P]
            if kw == 0:
                s = jnp.where(mask_l, 0.0, s)
            elif kw == 2:
                s = jnp.where(mask_r, 0.0, s)
            taps.append(s)
    patch = jnp.concatenate(taps, axis=0)
    return jnp.dot(w, patch, preferred_element_type=jnp.float32)


def _pool_bias_relu(conv, b, scr, *, C, H, W):
    scr[...] = conv.reshape(C, H, W)
    hp = jnp.maximum(scr[:, 0::2, :], scr[:, 1::2, :])
    flat = hp.reshape(C * (H // 2), W)
    row = jax.lax.broadcasted_iota(jnp.int32, (W, W // 2), 0)
    col2 = 2 * jax.lax.broadcasted_iota(jnp.int32, (W, W // 2), 1)
    sel_e = (row == col2).astype(jnp.float32)
    sel_o = (row == col2 + 1).astype(jnp.float32)
    wp = jnp.maximum(
        jnp.dot(flat, sel_e, preferred_element_type=jnp.float32),
        jnp.dot(flat, sel_o, preferred_element_type=jnp.float32))
    y = jnp.maximum(wp.reshape(C, H // 2, W // 2) + b.reshape(C, 1, 1), 0.0)
    return y.reshape(C, (H // 2) * (W // 2))


def _convs_kernel(x_ref, w1_ref, b1_ref, w2_ref, b2_ref, o_ref,
                  xpad_ref, ypad_ref, c1scr_ref, c2scr_ref):
    xpad_ref[:, :2 * _W1] = jnp.zeros((_C0, 2 * _W1), jnp.float32)
    xpad_ref[:, 2 * _W1 + _P1:] = jnp.zeros((_C0, 2 * _W1), jnp.float32)
    xpad_ref[:, 2 * _W1:2 * _W1 + _P1] = x_ref[0]
    conv1 = _im2col_dot(xpad_ref, w1_ref[...], Cin=_C0, W=_W1, P=_P1)
    y1 = _pool_bias_relu(conv1, b1_ref[...], c1scr_ref, C=_C1, H=_H1, W=_W1)

    ypad_ref[:, :2 * _W2] = jnp.zeros((_C1, 2 * _W2), jnp.float32)
    ypad_ref[:, 2 * _W2 + _P2:] = jnp.zeros((_C1, 2 * _W2), jnp.float32)
    ypad_ref[:, 2 * _W2:2 * _W2 + _P2] = y1
    conv2 = _im2col_dot(ypad_ref, w2_ref[...], Cin=_C1, W=_W2, P=_P2)
    o_ref[0] = _pool_bias_relu(conv2, b2_ref[...], c2scr_ref,
                               C=_C2, H=_H2, W=_W2)


def _fused_convs(x_flat, w1, b1, w2, b2):
    B = x_flat.shape[0]
    return pl.pallas_call(
        _convs_kernel,
        out_shape=jax.ShapeDtypeStruct((B, _C2, _P3), jnp.float32),
        grid=(B,),
        in_specs=[
            pl.BlockSpec((1, _C0, _P1), lambda b: (b, 0, 0)),
            pl.BlockSpec((_C1, 9 * _C0), lambda b: (0, 0)),
            pl.BlockSpec((_C1, 1), lambda b: (0, 0)),
            pl.BlockSpec((_C2, 9 * _C1), lambda b: (0, 0)),
            pl.BlockSpec((_C2, 1), lambda b: (0, 0)),
        ],
        out_specs=pl.BlockSpec((1, _C2, _P3), lambda b: (b, 0, 0)),
        scratch_shapes=[
            pltpu.VMEM((_C0, _P1 + 4 * _W1), jnp.float32),
            pltpu.VMEM((_C1, _P2 + 4 * _W2), jnp.float32),
            pltpu.VMEM((_C1, _H1, _W1), jnp.float32),
            pltpu.VMEM((_C2, _H2, _W2), jnp.float32),
        ],
        compiler_params=pltpu.CompilerParams(
            dimension_semantics=("arbitrary",)),
    )(x_flat, w1, b1, w2, b2)


def _fc_kernel(x_ref, w1_ref, b1_ref, w2_ref, b2_ref, w3_ref, b3_ref,
               o_ref, acc_ref):
    k = pl.program_id(0)

    @pl.when(k == 0)
    def _():
        acc_ref[...] = jnp.zeros_like(acc_ref)

    acc_ref[...] += jnp.dot(x_ref[...], w1_ref[...],
                            preferred_element_type=jnp.float32)

    @pl.when(k == pl.num_programs(0) - 1)
    def _():
        h1 = acc_ref[...] + b1_ref[...]
        h2 = jnp.dot(h1, w2_ref[...],
                     preferred_element_type=jnp.float32) + b2_ref[...]
        o_ref[...] = (jnp.dot(h2, w3_ref[...],
                              preferred_element_type=jnp.float32)
                      + b3_ref[...]).astype(o_ref.dtype)


def _fc_head(x_flat, w1, b1, w2, b2, w3, b3, *, tk=8192):
    B, K = x_flat.shape
    H1, H2, NC = w1.shape[1], w2.shape[1], w3.shape[1]
    return pl.pallas_call(
        _fc_kernel,
        out_shape=jax.ShapeDtypeStruct((B, NC), jnp.float32),
        grid=(K // tk,),
        in_specs=[
            pl.BlockSpec((B, tk), lambda k: (0, k)),
            pl.BlockSpec((tk, H1), lambda k: (k, 0)),
            pl.BlockSpec((1, H1), lambda k: (0, 0)),
            pl.BlockSpec((H1, H2), lambda k: (0, 0)),
            pl.BlockSpec((1, H2), lambda k: (0, 0)),
            pl.BlockSpec((H2, NC), lambda k: (0, 0)),
            pl.BlockSpec((1, NC), lambda k: (0, 0)),
        ],
        out_specs=pl.BlockSpec((B, NC), lambda k: (0, 0)),
        scratch_shapes=[pltpu.VMEM((B, H1), jnp.float32)],
        compiler_params=pltpu.CompilerParams(
            dimension_semantics=("arbitrary",)),
    )(x_flat, w1, b1, w2, b2, w3, b3)


@jax.jit
def kernel(x, conv1_w, conv1_b, conv2_w, conv2_b,
           fc1_w, fc1_b, fc2_w, fc2_b, fc3_w, fc3_b):
    B = x.shape[0]
    x_flat = x.astype(jnp.float32).reshape(B, _C0, _P1)
    y2 = _fused_convs(x_flat, conv1_w, conv1_b, conv2_w, conv2_b)
    flat = y2.reshape(B, _C2 * _P3)
    return _fc_head(flat, fc1_w, fc1_b, fc2_w, fc2_b, fc3_w, fc3_b)
